# baseline (speedup 1.0000x reference)
import functools

import jax
import jax.numpy as jnp
import numpy as np
from jax import lax
from jax.experimental import pallas as pl
from jax.experimental.pallas import tpu as pltpu
from jax.experimental.pallas import tpu_sc as plsc

N_NODES = 10000
N_EDGES = 320000
D_IN = 128
H = 8
D = 16
HD = H * D
AXW = 16

NC = 2
NS = 16
NW = NC * NS
EPW = N_EDGES // NW
CHUNK = 128
NFULL = EPW // CHUNK
TAIL = EPW - NFULL * CHUNK
C2 = 64
NF2 = EPW // C2
TAIL2 = EPW - NF2 * C2
NPB = 624
REM_BASE = NS * NPB
REM = N_NODES - REM_BASE
ZROWS = 8

_ONES_HD = np.concatenate(
    [np.kron(np.eye(H), np.ones((D, 1))), np.zeros((HD, AXW - H))],
    axis=1).astype(np.float32)
_EXPAND = np.concatenate(
    [np.kron(np.eye(H), np.ones((1, D))), np.zeros((AXW - H, HD))],
    axis=0).astype(np.float32)

_BN = 1000
_BE = 1000


def _proj_body(x_ref, wq_ref, wk_ref, wv_ref, q_ref, k_ref, v_ref):
    xb = x_ref[...]
    q_ref[...] = jnp.dot(xb, wq_ref[...], preferred_element_type=jnp.float32)
    k_ref[...] = jnp.dot(xb, wk_ref[...], preferred_element_type=jnp.float32)
    v_ref[...] = jnp.dot(xb, wv_ref[...], preferred_element_type=jnp.float32)


def _project_qkv(x, WQ, WK, WV):
    bs_w = pl.BlockSpec((D_IN, HD), lambda i: (0, 0))
    return pl.pallas_call(
        _proj_body,
        grid=(N_NODES // _BN,),
        in_specs=[pl.BlockSpec((_BN, D_IN), lambda i: (i, 0)), bs_w, bs_w, bs_w],
        out_specs=[pl.BlockSpec((_BN, HD), lambda i: (i, 0))] * 3,
        out_shape=[jax.ShapeDtypeStruct((N_NODES, HD), jnp.float32)] * 3,
    )(x, WQ, WK, WV)


def _edge_body(ea_ref, g_ref, we_ref, ones_ref, eout_ref, ax_ref):
    t = jnp.dot(ea_ref[...], we_ref[...], preferred_element_type=jnp.float32)
    eo = g_ref[...] * t
    eout_ref[...] = eo
    s = jnp.dot(eo, ones_ref[...], preferred_element_type=jnp.float32)
    ax_ref[...] = jnp.exp(jnp.clip(s, -5.0, 5.0))


def _edge_stage(edge_attr, g, WE):
    return pl.pallas_call(
        _edge_body,
        grid=(N_EDGES // _BE,),
        in_specs=[
            pl.BlockSpec((_BE, D_IN), lambda i: (i, 0)),
            pl.BlockSpec((_BE, HD), lambda i: (i, 0)),
            pl.BlockSpec((D_IN, HD), lambda i: (0, 0)),
            pl.BlockSpec((HD, AXW), lambda i: (0, 0)),
        ],
        out_specs=[
            pl.BlockSpec((_BE, HD), lambda i: (i, 0)),
            pl.BlockSpec((_BE, AXW), lambda i: (i, 0)),
        ],
        out_shape=[
            jax.ShapeDtypeStruct((N_EDGES, HD), jnp.float32),
            jax.ShapeDtypeStruct((N_EDGES, AXW), jnp.float32),
        ],
    )(edge_attr, g, WE, _ONES_HD)


def _final_body(a_ref, b_ref, za_ref, zb_ref, exp_ref, h_ref):
    s = a_ref[...] + b_ref[...]
    z = za_ref[...] + zb_ref[...]
    zfull = jnp.dot(z, exp_ref[...], preferred_element_type=jnp.float32)
    h_ref[...] = s / (zfull + 1e-6)


def _finalize(pv, pz):
    nb = N_NODES // _BN
    return pl.pallas_call(
        _final_body,
        grid=(nb,),
        in_specs=[
            pl.BlockSpec((_BN, HD), lambda i: (i, 0)),
            pl.BlockSpec((_BN, HD), lambda i: (i + nb, 0)),
            pl.BlockSpec((_BN, AXW), lambda i: (i, 0)),
            pl.BlockSpec((_BN, AXW), lambda i: (i + nb, 0)),
            pl.BlockSpec((AXW, HD), lambda i: (0, 0)),
        ],
        out_specs=pl.BlockSpec((_BN, HD), lambda i: (i, 0)),
        out_shape=jax.ShapeDtypeStruct((N_NODES, HD), jnp.float32),
    )(pv, pv, pz, pz, _EXPAND)


def _sc_gather_alpha_body(k_hbm, q_hbm, row_hbm, col_hbm, g_hbm,
                          ridx, cidx, krows, qrows, gout,
                          ridx_t, cidx_t, krows_t, qrows_t, gout_t, sem):
    wid = lax.axis_index("s") * NC + lax.axis_index("c")
    base0 = wid * EPW

    def do_chunk(base, cnt, ridx, cidx, krows, qrows, gout):
        pltpu.sync_copy(row_hbm.at[pl.ds(base, cnt)], ridx)
        pltpu.sync_copy(col_hbm.at[pl.ds(base, cnt)], cidx)
        cp1 = pltpu.async_copy(k_hbm.at[ridx], krows, sem)
        cp2 = pltpu.async_copy(q_hbm.at[cidx], qrows, sem)
        cp1.wait()
        cp2.wait()

        def row_body(e, carry):
            for h in range(H):
                sl = pl.ds(h * D, D)
                gout[e, sl] = jnp.clip(krows[e, sl] * qrows[e, sl] * 0.25,
                                       -5.0, 5.0)
            return carry

        lax.fori_loop(0, cnt, row_body, 0, unroll=2)
        pltpu.sync_copy(gout, g_hbm.at[pl.ds(base, cnt)])

    def chunk_body(j, carry):
        do_chunk(base0 + j * CHUNK, CHUNK, ridx, cidx, krows, qrows, gout)
        return carry

    lax.fori_loop(0, NFULL, chunk_body, 0)
    do_chunk(base0 + NFULL * CHUNK, TAIL, ridx_t, cidx_t, krows_t, qrows_t,
             gout_t)


def _sc_aggregate_body(v_hbm, ax_hbm, row_hbm, col_hbm, pv_hbm, pz_hbm,
                       ridx, cidx, vrows, axrows, cv, azr, scidx,
                       ridx_t, cidx_t, vrows_t, axrows_t, cv_t, azr_t, scidx_t,
                       zv, zz, accv, accz, sem, sems):
    cid = lax.axis_index("c")
    sid = lax.axis_index("s")
    wid = sid * NC + cid
    base0 = wid * EPW
    nbase = sid * NPB

    def zrow_v(r, carry):
        for j in range(HD // D):
            zv[r, pl.ds(j * D, D)] = jnp.zeros((D,), jnp.float32)
        zz[r, :] = jnp.zeros((AXW,), jnp.float32)
        return carry

    lax.fori_loop(0, ZROWS, zrow_v, 0)

    def zcopy(t, carry):
        pltpu.sync_copy(zv, accv.at[pl.ds(nbase + t * ZROWS, ZROWS)])
        pltpu.sync_copy(zz, accz.at[pl.ds(nbase + t * ZROWS, ZROWS)])
        return carry

    lax.fori_loop(0, NPB // ZROWS, zcopy, 0)

    @pl.when(sid == NS - 1)
    def _zero_rem():
        for t in range(REM // ZROWS):
            pltpu.sync_copy(zv, accv.at[pl.ds(REM_BASE + t * ZROWS, ZROWS)])
            pltpu.sync_copy(zz, accz.at[pl.ds(REM_BASE + t * ZROWS, ZROWS)])

    plsc.subcore_barrier()

    def do_chunk(base, cnt, ridx, cidx, vrows, axrows, cvb, azrb, sci,
                 dodrain):
        pltpu.sync_copy(row_hbm.at[pl.ds(base, cnt)], ridx)
        pltpu.sync_copy(col_hbm.at[pl.ds(base, cnt)], cidx)
        cp1 = pltpu.async_copy(v_hbm.at[ridx], vrows, sem)
        pltpu.sync_copy(ax_hbm.at[pl.ds(base, cnt)], axrows)
        cp1.wait()
        dodrain()

        def row_body(e, carry):
            ax16 = axrows[e, :]
            azrb[e, :] = ax16
            for h in range(H):
                sl = pl.ds(h * D, D)
                cvb[e, sl] = vrows[e, sl] * ax16[h]
            return carry

        lax.fori_loop(0, cnt, row_body, 0, unroll=2)
        for t in range(cnt // D):
            sci[pl.ds(t * D, D)] = cidx[pl.ds(t * D, D)]
        pltpu.async_copy(cvb, accv.at[sci], sems, add=True)
        pltpu.async_copy(azrb, accz.at[sci], sems, add=True)

    def drain_full():
        pltpu.make_async_copy(cv, accv.at[scidx], sems).wait()
        pltpu.make_async_copy(azr, accz.at[scidx], sems).wait()

    def chunk_body(j, carry):
        def dodrain():
            @pl.when(j > 0)
            def _():
                drain_full()

        do_chunk(base0 + j * CHUNK, CHUNK, ridx, cidx, vrows, axrows, cv,
                 azr, scidx, dodrain)
        return carry

    lax.fori_loop(0, NFULL, chunk_body, 0)
    do_chunk(base0 + NFULL * CHUNK, TAIL, ridx_t, cidx_t, vrows_t, axrows_t,
             cv_t, azr_t, scidx_t, drain_full)
    pltpu.make_async_copy(cv_t, accv.at[scidx_t], sems).wait()
    pltpu.make_async_copy(azr_t, accz.at[scidx_t], sems).wait()

    plsc.subcore_barrier()
    pltpu.sync_copy(accv.at[pl.ds(nbase, NPB)],
                    pv_hbm.at[pl.ds(cid * N_NODES + nbase, NPB)])
    pltpu.sync_copy(accz.at[pl.ds(nbase, NPB)],
                    pz_hbm.at[pl.ds(cid * N_NODES + nbase, NPB)])

    @pl.when(sid == NS - 1)
    def _dump_rem():
        pltpu.sync_copy(accv.at[pl.ds(REM_BASE, REM)],
                        pv_hbm.at[pl.ds(cid * N_NODES + REM_BASE, REM)])
        pltpu.sync_copy(accz.at[pl.ds(REM_BASE, REM)],
                        pz_hbm.at[pl.ds(cid * N_NODES + REM_BASE, REM)])


@functools.cache
def _sc_kernels():
    mesh = plsc.VectorSubcoreMesh(core_axis_name="c", subcore_axis_name="s",
                                  num_cores=NC, num_subcores=NS)
    scp = pltpu.CompilerParams(use_tc_tiling_on_sc=False)
    gather_alpha = pl.kernel(
        _sc_gather_alpha_body,
        mesh=mesh,
        compiler_params=scp,
        out_type=jax.ShapeDtypeStruct((N_EDGES, HD), jnp.float32),
        scratch_types=[
            pltpu.VMEM((CHUNK,), jnp.int32),
            pltpu.VMEM((CHUNK,), jnp.int32),
            pltpu.VMEM((CHUNK, HD), jnp.float32),
            pltpu.VMEM((CHUNK, HD), jnp.float32),
            pltpu.VMEM((CHUNK, HD), jnp.float32),
            pltpu.VMEM((TAIL,), jnp.int32),
            pltpu.VMEM((TAIL,), jnp.int32),
            pltpu.VMEM((TAIL, HD), jnp.float32),
            pltpu.VMEM((TAIL, HD), jnp.float32),
            pltpu.VMEM((TAIL, HD), jnp.float32),
            pltpu.SemaphoreType.DMA,
        ],
    )
    aggregate = pl.kernel(
        _sc_aggregate_body,
        mesh=mesh,
        compiler_params=scp,
        out_type=(
            jax.ShapeDtypeStruct((NC * N_NODES, HD), jnp.float32),
            jax.ShapeDtypeStruct((NC * N_NODES, AXW), jnp.float32),
        ),
        scratch_types=[
            pltpu.VMEM((CHUNK,), jnp.int32),
            pltpu.VMEM((CHUNK,), jnp.int32),
            pltpu.VMEM((CHUNK, HD), jnp.float32),
            pltpu.VMEM((CHUNK, AXW), jnp.float32),
            pltpu.VMEM((CHUNK, HD), jnp.float32),
            pltpu.VMEM((CHUNK, AXW), jnp.float32),
            pltpu.VMEM((CHUNK,), jnp.int32),
            pltpu.VMEM((TAIL,), jnp.int32),
            pltpu.VMEM((TAIL,), jnp.int32),
            pltpu.VMEM((TAIL, HD), jnp.float32),
            pltpu.VMEM((TAIL, AXW), jnp.float32),
            pltpu.VMEM((TAIL, HD), jnp.float32),
            pltpu.VMEM((TAIL, AXW), jnp.float32),
            pltpu.VMEM((TAIL,), jnp.int32),
            pltpu.VMEM((ZROWS, HD), jnp.float32),
            pltpu.VMEM((ZROWS, AXW), jnp.float32),
            pltpu.VMEM_SHARED((N_NODES, HD), jnp.float32),
            pltpu.VMEM_SHARED((N_NODES, AXW), jnp.float32),
            pltpu.SemaphoreType.DMA,
            pltpu.SemaphoreType.DMA,
        ],
    )
    return gather_alpha, aggregate


def kernel(x, edge_attr, edge_index, WQ, WK, WV, WE):
    gather_alpha, aggregate = _sc_kernels()
    row = edge_index[0]
    col = edge_index[1]
    q, k, v = _project_qkv(x, WQ, WK, WV)
    g = gather_alpha(k, q, row, col)
    e_out, ax = _edge_stage(edge_attr, g, WE)
    pv, pz = aggregate(v, ax, row, col)
    h = _finalize(pv, pz)
    return (h.reshape(N_NODES, H, D), e_out.reshape(N_EDGES, H, D))


# speedup vs baseline: 1.5002x; 1.5002x over previous
import functools

import jax
import jax.numpy as jnp
import numpy as np
from jax import lax
from jax.experimental import pallas as pl
from jax.experimental.pallas import tpu as pltpu
from jax.experimental.pallas import tpu_sc as plsc

N_NODES = 10000
N_EDGES = 320000
D_IN = 128
H = 8
D = 16
HD = H * D
AXW = 16

NC = 2
NS = 16
NW = NC * NS
EPW = N_EDGES // NW
CHUNK = 128
NFULL = EPW // CHUNK
TAIL = EPW - NFULL * CHUNK
C2 = 64
NF2 = EPW // C2
TAIL2 = EPW - NF2 * C2
NPB = 624
REM_BASE = NS * NPB
REM = N_NODES - REM_BASE
ZROWS = 8

_ONES_HD = np.concatenate(
    [np.kron(np.eye(H), np.ones((D, 1))), np.zeros((HD, AXW - H))],
    axis=1).astype(np.float32)
_EXPAND = np.concatenate(
    [np.kron(np.eye(H), np.ones((1, D))), np.zeros((AXW - H, HD))],
    axis=0).astype(np.float32)

_BN = 1000
_BE = 1000


def _proj_body(x_ref, wq_ref, wk_ref, wv_ref, q_ref, k_ref, v_ref):
    xb = x_ref[...]
    q_ref[...] = jnp.dot(xb, wq_ref[...], preferred_element_type=jnp.float32)
    k_ref[...] = jnp.dot(xb, wk_ref[...], preferred_element_type=jnp.float32)
    v_ref[...] = jnp.dot(xb, wv_ref[...], preferred_element_type=jnp.float32)


def _project_qkv(x, WQ, WK, WV):
    bs_w = pl.BlockSpec((D_IN, HD), lambda i: (0, 0))
    return pl.pallas_call(
        _proj_body,
        grid=(N_NODES // _BN,),
        in_specs=[pl.BlockSpec((_BN, D_IN), lambda i: (i, 0)), bs_w, bs_w, bs_w],
        out_specs=[pl.BlockSpec((_BN, HD), lambda i: (i, 0))] * 3,
        out_shape=[jax.ShapeDtypeStruct((N_NODES, HD), jnp.float32)] * 3,
    )(x, WQ, WK, WV)


def _edge_body(ea_ref, g_ref, we_ref, ones_ref, eout_ref, ax_ref):
    t = jnp.dot(ea_ref[...], we_ref[...], preferred_element_type=jnp.float32)
    eo = g_ref[...] * t
    eout_ref[...] = eo
    s = jnp.dot(eo, ones_ref[...], preferred_element_type=jnp.float32)
    ax_ref[...] = jnp.exp(jnp.clip(s, -5.0, 5.0))


def _edge_stage(edge_attr, g, WE):
    return pl.pallas_call(
        _edge_body,
        grid=(N_EDGES // _BE,),
        in_specs=[
            pl.BlockSpec((_BE, D_IN), lambda i: (i, 0)),
            pl.BlockSpec((_BE, HD), lambda i: (i, 0)),
            pl.BlockSpec((D_IN, HD), lambda i: (0, 0)),
            pl.BlockSpec((HD, AXW), lambda i: (0, 0)),
        ],
        out_specs=[
            pl.BlockSpec((_BE, HD), lambda i: (i, 0)),
            pl.BlockSpec((_BE, AXW), lambda i: (i, 0)),
        ],
        out_shape=[
            jax.ShapeDtypeStruct((N_EDGES, HD), jnp.float32),
            jax.ShapeDtypeStruct((N_EDGES, AXW), jnp.float32),
        ],
    )(edge_attr, g, WE, _ONES_HD)


def _final_body(a_ref, b_ref, za_ref, zb_ref, exp_ref, h_ref):
    s = a_ref[...] + b_ref[...]
    z = za_ref[...] + zb_ref[...]
    zfull = jnp.dot(z, exp_ref[...], preferred_element_type=jnp.float32)
    h_ref[...] = s / (zfull + 1e-6)


def _finalize(pv, pz):
    nb = N_NODES // _BN
    return pl.pallas_call(
        _final_body,
        grid=(nb,),
        in_specs=[
            pl.BlockSpec((_BN, HD), lambda i: (i, 0)),
            pl.BlockSpec((_BN, HD), lambda i: (i + nb, 0)),
            pl.BlockSpec((_BN, AXW), lambda i: (i, 0)),
            pl.BlockSpec((_BN, AXW), lambda i: (i + nb, 0)),
            pl.BlockSpec((AXW, HD), lambda i: (0, 0)),
        ],
        out_specs=pl.BlockSpec((_BN, HD), lambda i: (i, 0)),
        out_shape=jax.ShapeDtypeStruct((N_NODES, HD), jnp.float32),
    )(pv, pv, pz, pz, _EXPAND)


def _sc_gather_alpha_body(k_hbm, q_hbm, row_hbm, col_hbm, g_hbm,
                          ridx, cidx, krows, qrows, gout,
                          ridx_t, cidx_t, krows_t, qrows_t, gout_t, sem):
    wid = lax.axis_index("s") * NC + lax.axis_index("c")
    base0 = wid * EPW

    def do_chunk(base, cnt, ridx, cidx, krows, qrows, gout):
        pltpu.sync_copy(row_hbm.at[pl.ds(base, cnt)], ridx)
        pltpu.sync_copy(col_hbm.at[pl.ds(base, cnt)], cidx)
        cp1 = pltpu.async_copy(k_hbm.at[ridx], krows, sem)
        cp2 = pltpu.async_copy(q_hbm.at[cidx], qrows, sem)
        cp1.wait()
        cp2.wait()

        def row_body(e, carry):
            for h in range(H):
                sl = pl.ds(h * D, D)
                gout[e, sl] = jnp.clip(krows[e, sl] * qrows[e, sl] * 0.25,
                                       -5.0, 5.0)
            return carry

        lax.fori_loop(0, cnt, row_body, 0)
        pltpu.sync_copy(gout, g_hbm.at[pl.ds(base, cnt)])

    def chunk_body(j, carry):
        do_chunk(base0 + j * CHUNK, CHUNK, ridx, cidx, krows, qrows, gout)
        return carry

    lax.fori_loop(0, NFULL, chunk_body, 0)
    do_chunk(base0 + NFULL * CHUNK, TAIL, ridx_t, cidx_t, krows_t, qrows_t,
             gout_t)


def _sc_aggregate_body(v_hbm, ax_hbm, row_hbm, col_hbm, pv_hbm, pz_hbm,
                       ridx, cidx, vrows, axrows, cv,
                       ridx_t, cidx_t, vrows_t, axrows_t, cv_t,
                       zv, zz, accv, accz, sem):
    cid = lax.axis_index("c")
    sid = lax.axis_index("s")
    wid = sid * NC + cid
    base0 = wid * EPW
    nbase = sid * NPB

    def zrow_v(r, carry):
        for j in range(HD // D):
            zv[r, pl.ds(j * D, D)] = jnp.zeros((D,), jnp.float32)
        zz[r, :] = jnp.zeros((AXW,), jnp.float32)
        return carry

    lax.fori_loop(0, ZROWS, zrow_v, 0)

    def zcopy(t, carry):
        pltpu.sync_copy(zv, accv.at[pl.ds(nbase + t * ZROWS, ZROWS)])
        pltpu.sync_copy(zz, accz.at[pl.ds(nbase + t * ZROWS, ZROWS)])
        return carry

    lax.fori_loop(0, NPB // ZROWS, zcopy, 0)

    @pl.when(sid == NS - 1)
    def _zero_rem():
        for t in range(REM // ZROWS):
            pltpu.sync_copy(zv, accv.at[pl.ds(REM_BASE + t * ZROWS, ZROWS)])
            pltpu.sync_copy(zz, accz.at[pl.ds(REM_BASE + t * ZROWS, ZROWS)])

    plsc.subcore_barrier()

    def do_chunk(base, cnt, ridx, cidx, vrows, axrows, cv):
        pltpu.sync_copy(row_hbm.at[pl.ds(base, cnt)], ridx)
        pltpu.sync_copy(col_hbm.at[pl.ds(base, cnt)], cidx)
        cp1 = pltpu.async_copy(v_hbm.at[ridx], vrows, sem)
        pltpu.sync_copy(ax_hbm.at[pl.ds(base, cnt)], axrows)
        cp1.wait()

        def row_body(e, carry):
            ax16 = axrows[e, :]
            for h in range(H):
                sl = pl.ds(h * D, D)
                cv[e, sl] = vrows[e, sl] * ax16[h]
            return carry

        lax.fori_loop(0, cnt, row_body, 0)
        pltpu.sync_copy(cv, accv.at[cidx], add=True)
        pltpu.sync_copy(axrows, accz.at[cidx], add=True)

    def chunk_body(j, carry):
        do_chunk(base0 + j * C2, C2, ridx, cidx, vrows, axrows, cv)
        return carry

    lax.fori_loop(0, NF2, chunk_body, 0)
    do_chunk(base0 + NF2 * C2, TAIL2, ridx_t, cidx_t, vrows_t, axrows_t,
             cv_t)

    plsc.subcore_barrier()
    pltpu.sync_copy(accv.at[pl.ds(nbase, NPB)],
                    pv_hbm.at[pl.ds(cid * N_NODES + nbase, NPB)])
    pltpu.sync_copy(accz.at[pl.ds(nbase, NPB)],
                    pz_hbm.at[pl.ds(cid * N_NODES + nbase, NPB)])

    @pl.when(sid == NS - 1)
    def _dump_rem():
        pltpu.sync_copy(accv.at[pl.ds(REM_BASE, REM)],
                        pv_hbm.at[pl.ds(cid * N_NODES + REM_BASE, REM)])
        pltpu.sync_copy(accz.at[pl.ds(REM_BASE, REM)],
                        pz_hbm.at[pl.ds(cid * N_NODES + REM_BASE, REM)])


@functools.cache
def _sc_kernels():
    mesh = plsc.VectorSubcoreMesh(core_axis_name="c", subcore_axis_name="s",
                                  num_cores=NC, num_subcores=NS)
    scp = pltpu.CompilerParams(use_tc_tiling_on_sc=False)
    gather_alpha = pl.kernel(
        _sc_gather_alpha_body,
        mesh=mesh,
        compiler_params=scp,
        out_type=jax.ShapeDtypeStruct((N_EDGES, HD), jnp.float32),
        scratch_types=[
            pltpu.VMEM((CHUNK,), jnp.int32),
            pltpu.VMEM((CHUNK,), jnp.int32),
            pltpu.VMEM((CHUNK, HD), jnp.float32),
            pltpu.VMEM((CHUNK, HD), jnp.float32),
            pltpu.VMEM((CHUNK, HD), jnp.float32),
            pltpu.VMEM((TAIL,), jnp.int32),
            pltpu.VMEM((TAIL,), jnp.int32),
            pltpu.VMEM((TAIL, HD), jnp.float32),
            pltpu.VMEM((TAIL, HD), jnp.float32),
            pltpu.VMEM((TAIL, HD), jnp.float32),
            pltpu.SemaphoreType.DMA,
        ],
    )
    aggregate = pl.kernel(
        _sc_aggregate_body,
        mesh=mesh,
        compiler_params=scp,
        out_type=(
            jax.ShapeDtypeStruct((NC * N_NODES, HD), jnp.float32),
            jax.ShapeDtypeStruct((NC * N_NODES, AXW), jnp.float32),
        ),
        scratch_types=[
            pltpu.VMEM((C2,), jnp.int32),
            pltpu.VMEM((C2,), jnp.int32),
            pltpu.VMEM((C2, HD), jnp.float32),
            pltpu.VMEM((C2, AXW), jnp.float32),
            pltpu.VMEM((C2, HD), jnp.float32),
            pltpu.VMEM((TAIL2,), jnp.int32),
            pltpu.VMEM((TAIL2,), jnp.int32),
            pltpu.VMEM((TAIL2, HD), jnp.float32),
            pltpu.VMEM((TAIL2, AXW), jnp.float32),
            pltpu.VMEM((TAIL2, HD), jnp.float32),
            pltpu.VMEM((ZROWS, HD), jnp.float32),
            pltpu.VMEM((ZROWS, AXW), jnp.float32),
            pltpu.VMEM_SHARED((N_NODES, HD), jnp.float32),
            pltpu.VMEM_SHARED((N_NODES, AXW), jnp.float32),
            pltpu.SemaphoreType.DMA,
        ],
    )
    return gather_alpha, aggregate


def kernel(x, edge_attr, edge_index, WQ, WK, WV, WE):
    gather_alpha, aggregate = _sc_kernels()
    row = edge_index[0]
    col = edge_index[1]
    q, k, v = _project_qkv(x, WQ, WK, WV)
    g = gather_alpha(k, q, row, col)
    e_out, ax = _edge_stage(edge_attr, g, WE)
    pv, pz = aggregate(v, ax, row, col)
    h = _finalize(pv, pz)
    return (h.reshape(N_NODES, H, D), e_out.reshape(N_EDGES, H, D))
